# baseline (speedup 1.0000x reference)
import functools

import jax
import jax.numpy as jnp
from jax import lax
from jax.experimental import pallas as pl
from jax.experimental.pallas import tpu as pltpu
from jax.experimental.pallas import tpu_sc as plsc

N = 10000
E = 320000
D = 128

NC = 2
NS = 16
NW = NC * NS
EP = E // NW
EC = 1000
CH = 40
NCHB = EC // CH
NQUAD = NCHB // 4
NGRP = EC // 16
REM = EC - NGRP * 16

ROWS_PER_TILE = 640


def _proj_body(x_ref, w_ref, a_ref, b_ref, sf_ref, f_ref):
    sf = jnp.dot(x_ref[...], w_ref[...], preferred_element_type=jnp.float32)
    sf_ref[...] = sf
    f_ref[...] = jnp.dot(sf, a_ref[...], preferred_element_type=jnp.float32) + b_ref[...]


def _tc_project(x, W, a_cat, b_cat):
    BR = 2000
    grid = (N // BR,)
    return pl.pallas_call(
        _proj_body,
        grid=grid,
        in_specs=[
            pl.BlockSpec((BR, D), lambda i: (i, 0)),
            pl.BlockSpec((D, D), lambda i: (0, 0)),
            pl.BlockSpec((D, 2), lambda i: (0, 0)),
            pl.BlockSpec((1, 2), lambda i: (0, 0)),
        ],
        out_specs=[
            pl.BlockSpec((BR, D), lambda i: (i, 0)),
            pl.BlockSpec((BR, 2), lambda i: (i, 0)),
        ],
        out_shape=[
            jax.ShapeDtypeStruct((N, D), jnp.float32),
            jax.ShapeDtypeStruct((N, 2), jnp.float32),
        ],
    )(x, W, a_cat, b_cat)


def _sc_edge_body(f12_hbm, src_hbm, dst_hbm, adj_hbm, sf_hbm,
                  denom_out, vals_out,
                  f12_v, src_v, dst_v, adj_v, e_v, denom_v, rows_v,
                  vals_sh, gs0, gs1, gs2, gs3, ss0, ss1, ss2, ss3):
    gss = [gs0, gs1, gs2, gs3]
    sss = [ss0, ss1, ss2, ss3]
    cid = lax.axis_index("c")
    sid = lax.axis_index("s")
    wid = sid * NC + cid
    base = wid * EP

    pltpu.sync_copy(f12_hbm, f12_v)

    zeros16 = jnp.zeros((16,), jnp.float32)

    def _zd(i, _):
        denom_v[pl.ds(i * 16, 16)] = zeros16
        return 0
    lax.fori_loop(0, N // 16, _zd, 0)

    def _zr(i, _):
        for b in range(4):
            for k in range(D // 16):
                rows_v[b, i, pl.ds(k * 16, 16)] = zeros16
        return 0
    lax.fori_loop(0, CH, _zr, 0)
    stripe = jnp.minimum(sid * ROWS_PER_TILE, N - ROWS_PER_TILE)
    for z in range(ROWS_PER_TILE // CH):
        pltpu.async_copy(rows_v.at[0],
                         vals_sh.at[pl.ds(stripe + z * CH, CH)], gs0)
    for z in range(ROWS_PER_TILE // CH):
        pltpu.make_async_copy(rows_v.at[0],
                              vals_sh.at[pl.ds(stripe + z * CH, CH)],
                              gs0).wait()
    plsc.subcore_barrier()

    def _block(blk, _):
        eb = base + blk * EC
        pltpu.sync_copy(src_hbm.at[pl.ds(eb, EC)], src_v)
        pltpu.sync_copy(dst_hbm.at[pl.ds(eb, EC)], dst_v)
        pltpu.sync_copy(adj_hbm.at[pl.ds(eb, EC)], adj_v)

        def _gidx(c):
            return dst_v.at[pl.ds(c * CH, CH)]

        def _sidx(c):
            return src_v.at[pl.ds(c * CH, CH)]

        def _gissue(c, b):
            pltpu.async_copy(sf_hbm.at[_gidx(c)], rows_v.at[b], gss[b])

        def _gwait(c, b):
            pltpu.make_async_copy(sf_hbm.at[_gidx(c)], rows_v.at[b],
                                  gss[b]).wait()

        def _sissue(c, b):
            pltpu.async_copy(rows_v.at[b], vals_sh.at[_sidx(c)], sss[b],
                             add=True)

        def _swait(c, b):
            pltpu.make_async_copy(rows_v.at[b], vals_sh.at[_sidx(c)],
                                  sss[b]).wait()

        _gissue(0, 0)
        _gissue(1, 1)

        def _e16(off, mask):
            s16 = src_v[pl.ds(off, 16)]
            d16 = dst_v[pl.ds(off, 16)]
            a16 = adj_v[pl.ds(off, 16)]
            ws = plsc.load_gather(f12_v, [s16])
            wd = plsc.load_gather(f12_v, [d16])
            f1s, _ = plsc.unpack(plsc.bitcast(ws, jnp.bfloat16),
                                 format=plsc.PackFormat.INTERLEAVED)
            _, f2d = plsc.unpack(plsc.bitcast(wd, jnp.bfloat16),
                                 format=plsc.PackFormat.INTERLEAVED)
            t = a16 * (f1s + f2d)
            lr = jnp.maximum(t, 0.2 * t)
            e16 = jnp.exp(lr)
            e_v[pl.ds(off, 16)] = e16
            plsc.addupdate_scatter(denom_v, [s16], e16, mask=mask)

        def _egrp(g, _):
            _e16(g * 16, None)
            return 0
        lax.fori_loop(0, NGRP, _egrp, 0)
        if REM:
            _e16(EC - 16, lax.iota(jnp.int32, 16) >= 16 - REM)

        def _proc(c, b):
            _gwait(c, b)
            for g in range(CH // 16):
                e16 = e_v[pl.ds(c * CH + g * 16, 16)]
                for j in range(16):
                    r = g * 16 + j
                    ej = e16[j]
                    for k in range(8):
                        rows_v[b, r, pl.ds(k * 16, 16)] = (
                            rows_v[b, r, pl.ds(k * 16, 16)] * ej)
            e16 = e_v[pl.ds(c * CH + CH - 16, 16)]
            for j in range(CH - (CH // 16) * 16):
                r = (CH // 16) * 16 + j
                ej = e16[16 - (CH - (CH // 16) * 16) + j]
                for k in range(8):
                    rows_v[b, r, pl.ds(k * 16, 16)] = (
                        rows_v[b, r, pl.ds(k * 16, 16)] * ej)

            _sissue(c, b)

        def _quad(t, _):
            for u in range(4):
                c = 4 * t + u
                pb = (u + 2) % 4
                if u in (0, 1):
                    @pl.when(t > 0)
                    def _():
                        _swait(c - 2, pb)
                    _gissue(c + 2, pb)
                elif u == 2:
                    _swait(c - 2, pb)
                    _gissue(c + 2, pb)
                else:
                    @pl.when(t < NQUAD - 1)
                    def _():
                        _swait(c - 2, pb)
                        _gissue(c + 2, pb)
                _proc(c, u)
            return 0
        lax.fori_loop(0, NQUAD, _quad, 0)
        _proc(NCHB - 1, 0)
        _swait(NCHB - 4, 1)
        _swait(NCHB - 3, 2)
        _swait(NCHB - 2, 3)
        _swait(NCHB - 1, 0)
        return 0
    lax.fori_loop(0, EP // EC, _block, 0)

    plsc.subcore_barrier()
    pltpu.sync_copy(denom_v, denom_out.at[pl.ds(wid * N, N)])
    pltpu.sync_copy(vals_sh.at[pl.ds(stripe, ROWS_PER_TILE)],
                    vals_out.at[cid, pl.ds(stripe, ROWS_PER_TILE)])


def _sc_edge(f12, src, dst, adj, sf):
    mesh = plsc.VectorSubcoreMesh(core_axis_name="c", subcore_axis_name="s",
                                  num_cores=NC)
    fn = functools.partial(
        pl.kernel,
        mesh=mesh,
        compiler_params=pltpu.CompilerParams(needs_layout_passes=False),
        out_type=[
            jax.ShapeDtypeStruct((NW * N,), jnp.float32),
            jax.ShapeDtypeStruct((NC, N, D), jnp.float32),
        ],
        scratch_types=[
            pltpu.VMEM((N,), jnp.int32),
            pltpu.VMEM((EC,), jnp.int32),
            pltpu.VMEM((EC,), jnp.int32),
            pltpu.VMEM((EC,), jnp.float32),
            pltpu.VMEM((EC,), jnp.float32),
            pltpu.VMEM((N,), jnp.float32),
            pltpu.VMEM((4, CH, D), jnp.float32),
            pltpu.VMEM_SHARED((N, D), jnp.float32),
            pltpu.SemaphoreType.DMA,
            pltpu.SemaphoreType.DMA,
            pltpu.SemaphoreType.DMA,
            pltpu.SemaphoreType.DMA,
            pltpu.SemaphoreType.DMA,
            pltpu.SemaphoreType.DMA,
            pltpu.SemaphoreType.DMA,
            pltpu.SemaphoreType.DMA,
        ],
    )(_sc_edge_body)
    return fn(f12, src, dst, adj, sf)


def _final_body(vals_ref, denom_ref, bias_ref, o_ref):
    v = vals_ref[0] + vals_ref[1]
    d = jnp.sum(denom_ref[...], axis=1)
    d = jnp.where(d > 0.0, d, 1.0)
    r = v / d[:, None] + bias_ref[...]
    o_ref[...] = jnp.where(r > 0.0, r, jnp.exp(jnp.minimum(r, 0.0)) - 1.0)


def _tc_finalize(vals_p, denom_t, bias):
    BR = 2000
    return pl.pallas_call(
        _final_body,
        grid=(N // BR,),
        in_specs=[
            pl.BlockSpec((NC, BR, D), lambda i: (0, i, 0)),
            pl.BlockSpec((BR, NW), lambda i: (i, 0)),
            pl.BlockSpec((1, D), lambda i: (0, 0)),
        ],
        out_specs=pl.BlockSpec((BR, D), lambda i: (i, 0)),
        out_shape=jax.ShapeDtypeStruct((N, D), jnp.float32),
    )(vals_p, denom_t, bias)


@jax.jit
def kernel(x, edge_index, adj_vals, W, a1, b1, a2, b2, bias):
    src = edge_index[0]
    dst = edge_index[1]
    a_cat = jnp.concatenate([a1, a2], axis=1)
    b_cat = jnp.stack([b1[0], b2[0]])[None, :]
    sf, f = _tc_project(x, W, a_cat, b_cat)
    f12 = jax.lax.bitcast_convert_type(f.astype(jnp.bfloat16), jnp.int32)
    denom_p, vals_p = _sc_edge(f12, src, dst, adj_vals, sf)
    denom_t = denom_p.reshape(NW, N).T
    return _tc_finalize(vals_p, denom_t, bias.reshape(1, D))


# speedup vs baseline: 1.2223x; 1.2223x over previous
import functools

import jax
import jax.numpy as jnp
from jax import lax
from jax.experimental import pallas as pl
from jax.experimental.pallas import tpu as pltpu
from jax.experimental.pallas import tpu_sc as plsc

N = 10000
E = 320000
D = 128

NC = 2
NS = 16
NW = NC * NS
EP = E // NW
EC = 1000
CH = 40
NCHB = EC // CH
NQUAD = NCHB // 4
NGRP = EC // 16
REM = EC - NGRP * 16

ROWS_PER_TILE = 640


def _proj_body(x_ref, w_ref, a_ref, b_ref, sf_ref, f_ref):
    sf = jnp.dot(x_ref[...], w_ref[...], preferred_element_type=jnp.float32)
    sf_ref[...] = sf
    f_ref[...] = jnp.dot(sf, a_ref[...], preferred_element_type=jnp.float32) + b_ref[...]


def _tc_project(x, W, a_cat, b_cat):
    BR = 2000
    grid = (N // BR,)
    return pl.pallas_call(
        _proj_body,
        grid=grid,
        in_specs=[
            pl.BlockSpec((BR, D), lambda i: (i, 0)),
            pl.BlockSpec((D, D), lambda i: (0, 0)),
            pl.BlockSpec((D, 2), lambda i: (0, 0)),
            pl.BlockSpec((1, 2), lambda i: (0, 0)),
        ],
        out_specs=[
            pl.BlockSpec((BR, D), lambda i: (i, 0)),
            pl.BlockSpec((BR, 2), lambda i: (i, 0)),
        ],
        out_shape=[
            jax.ShapeDtypeStruct((N, D), jnp.float32),
            jax.ShapeDtypeStruct((N, 2), jnp.float32),
        ],
    )(x, W, a_cat, b_cat)


def _sc_edge_body(f12_hbm, src_hbm, dst_hbm, adj_hbm, sf_hbm,
                  denom_out, vals_out,
                  f12_v, src_v, dst_v, adj_v, e_v, denom_v, rows_v,
                  vals_sh, gs0, gs1, gs2, gs3, ss0, ss1, ss2, ss3):
    gss = [gs0, gs1, gs2, gs3]
    sss = [ss0, ss1, ss2, ss3]
    cid = lax.axis_index("c")
    sid = lax.axis_index("s")
    wid = sid * NC + cid
    base = wid * EP

    pltpu.sync_copy(f12_hbm, f12_v)

    zeros16 = jnp.zeros((16,), jnp.float32)

    def _zd(i, _):
        denom_v[pl.ds(i * 16, 16)] = zeros16
        return 0
    lax.fori_loop(0, N // 16, _zd, 0)

    def _zr(i, _):
        for b in range(4):
            for k in range(D // 16):
                rows_v[b, i, pl.ds(k * 16, 16)] = zeros16
        return 0
    lax.fori_loop(0, CH, _zr, 0)
    stripe = jnp.minimum(sid * ROWS_PER_TILE, N - ROWS_PER_TILE)
    for z in range(ROWS_PER_TILE // CH):
        pltpu.async_copy(rows_v.at[0],
                         vals_sh.at[pl.ds(stripe + z * CH, CH)], gs0)
    for z in range(ROWS_PER_TILE // CH):
        pltpu.make_async_copy(rows_v.at[0],
                              vals_sh.at[pl.ds(stripe + z * CH, CH)],
                              gs0).wait()
    plsc.subcore_barrier()

    def _block(blk, _):
        eb = base + blk * EC
        pltpu.sync_copy(src_hbm.at[pl.ds(eb, EC)], src_v)
        pltpu.sync_copy(dst_hbm.at[pl.ds(eb, EC)], dst_v)
        pltpu.sync_copy(adj_hbm.at[pl.ds(eb, EC)], adj_v)

        def _gidx(c):
            return dst_v.at[pl.ds(c * CH, CH)]

        def _sidx(c):
            return src_v.at[pl.ds(c * CH, CH)]

        def _gissue(c, b):
            pltpu.async_copy(sf_hbm.at[_gidx(c)], rows_v.at[b], gss[b])

        def _gwait(c, b):
            pltpu.make_async_copy(sf_hbm.at[_gidx(c)], rows_v.at[b],
                                  gss[b]).wait()

        def _sissue(c, b):
            pltpu.async_copy(rows_v.at[b], vals_sh.at[_sidx(c)], sss[b],
                             add=True)

        def _swait(c, b):
            pltpu.make_async_copy(rows_v.at[b], vals_sh.at[_sidx(c)],
                                  sss[b]).wait()

        _gissue(0, 0)
        _gissue(1, 1)

        def _e16(off, mask):
            s16 = src_v[pl.ds(off, 16)]
            d16 = dst_v[pl.ds(off, 16)]
            a16 = adj_v[pl.ds(off, 16)]
            ws = plsc.load_gather(f12_v, [s16])
            wd = plsc.load_gather(f12_v, [d16])
            f1s, _ = plsc.unpack(plsc.bitcast(ws, jnp.bfloat16),
                                 format=plsc.PackFormat.INTERLEAVED)
            _, f2d = plsc.unpack(plsc.bitcast(wd, jnp.bfloat16),
                                 format=plsc.PackFormat.INTERLEAVED)
            t = a16 * (f1s + f2d)
            lr = jnp.maximum(t, 0.2 * t)
            e16 = jnp.exp(lr)
            e_v[pl.ds(off, 16)] = e16
            plsc.addupdate_scatter(denom_v, [s16], e16, mask=mask)

        def _egrp(g, _):
            _e16(g * 16, None)
            return 0
        lax.fori_loop(0, NGRP, _egrp, 0)
        if REM:
            _e16(EC - 16, lax.iota(jnp.int32, 16) >= 16 - REM)

        def _proc(c, b):
            _gwait(c, b)

            def _grp(g, _):
                e16 = e_v[pl.ds(c * CH + g * 16, 16)]
                for j in range(16):
                    r = g * 16 + j
                    ej = e16[j]
                    for k in range(8):
                        rows_v[b, r, pl.ds(k * 16, 16)] = (
                            rows_v[b, r, pl.ds(k * 16, 16)] * ej)
                return 0
            lax.fori_loop(0, CH // 16, _grp, 0)
            e16 = e_v[pl.ds(c * CH + CH - 16, 16)]
            for j in range(CH - (CH // 16) * 16):
                r = (CH // 16) * 16 + j
                ej = e16[16 - (CH - (CH // 16) * 16) + j]
                for k in range(8):
                    rows_v[b, r, pl.ds(k * 16, 16)] = (
                        rows_v[b, r, pl.ds(k * 16, 16)] * ej)

            _sissue(c, b)

        def _quad(t, _):
            for u in range(4):
                c = 4 * t + u
                pb = (u + 2) % 4
                if u in (0, 1):
                    @pl.when(t > 0)
                    def _():
                        _swait(c - 2, pb)
                    _gissue(c + 2, pb)
                elif u == 2:
                    _swait(c - 2, pb)
                    _gissue(c + 2, pb)
                else:
                    @pl.when(t < NQUAD - 1)
                    def _():
                        _swait(c - 2, pb)
                        _gissue(c + 2, pb)
                _proc(c, u)
            return 0
        lax.fori_loop(0, NQUAD, _quad, 0)
        _proc(NCHB - 1, 0)
        _swait(NCHB - 4, 1)
        _swait(NCHB - 3, 2)
        _swait(NCHB - 2, 3)
        _swait(NCHB - 1, 0)
        return 0
    lax.fori_loop(0, EP // EC, _block, 0)

    plsc.subcore_barrier()
    pltpu.sync_copy(denom_v, denom_out.at[pl.ds(wid * N, N)])
    pltpu.sync_copy(vals_sh.at[pl.ds(stripe, ROWS_PER_TILE)],
                    vals_out.at[cid, pl.ds(stripe, ROWS_PER_TILE)])


def _sc_edge(f12, src, dst, adj, sf):
    mesh = plsc.VectorSubcoreMesh(core_axis_name="c", subcore_axis_name="s",
                                  num_cores=NC)
    fn = functools.partial(
        pl.kernel,
        mesh=mesh,
        compiler_params=pltpu.CompilerParams(needs_layout_passes=False),
        out_type=[
            jax.ShapeDtypeStruct((NW * N,), jnp.float32),
            jax.ShapeDtypeStruct((NC, N, D), jnp.float32),
        ],
        scratch_types=[
            pltpu.VMEM((N,), jnp.int32),
            pltpu.VMEM((EC,), jnp.int32),
            pltpu.VMEM((EC,), jnp.int32),
            pltpu.VMEM((EC,), jnp.float32),
            pltpu.VMEM((EC,), jnp.float32),
            pltpu.VMEM((N,), jnp.float32),
            pltpu.VMEM((4, CH, D), jnp.float32),
            pltpu.VMEM_SHARED((N, D), jnp.float32),
            pltpu.SemaphoreType.DMA,
            pltpu.SemaphoreType.DMA,
            pltpu.SemaphoreType.DMA,
            pltpu.SemaphoreType.DMA,
            pltpu.SemaphoreType.DMA,
            pltpu.SemaphoreType.DMA,
            pltpu.SemaphoreType.DMA,
            pltpu.SemaphoreType.DMA,
        ],
    )(_sc_edge_body)
    return fn(f12, src, dst, adj, sf)


def _final_body(vals_ref, denom_ref, bias_ref, o_ref):
    v = vals_ref[0] + vals_ref[1]
    d = jnp.sum(denom_ref[...], axis=1)
    d = jnp.where(d > 0.0, d, 1.0)
    r = v / d[:, None] + bias_ref[...]
    o_ref[...] = jnp.where(r > 0.0, r, jnp.exp(jnp.minimum(r, 0.0)) - 1.0)


def _tc_finalize(vals_p, denom_t, bias):
    BR = 2000
    return pl.pallas_call(
        _final_body,
        grid=(N // BR,),
        in_specs=[
            pl.BlockSpec((NC, BR, D), lambda i: (0, i, 0)),
            pl.BlockSpec((BR, NW), lambda i: (i, 0)),
            pl.BlockSpec((1, D), lambda i: (0, 0)),
        ],
        out_specs=pl.BlockSpec((BR, D), lambda i: (i, 0)),
        out_shape=jax.ShapeDtypeStruct((N, D), jnp.float32),
    )(vals_p, denom_t, bias)


@jax.jit
def kernel(x, edge_index, adj_vals, W, a1, b1, a2, b2, bias):
    src = edge_index[0]
    dst = edge_index[1]
    a_cat = jnp.concatenate([a1, a2], axis=1)
    b_cat = jnp.stack([b1[0], b2[0]])[None, :]
    sf, f = _tc_project(x, W, a_cat, b_cat)
    f12 = jax.lax.bitcast_convert_type(f.astype(jnp.bfloat16), jnp.int32)
    denom_p, vals_p = _sc_edge(f12, src, dst, adj_vals, sf)
    denom_t = denom_p.reshape(NW, N).T
    return _tc_finalize(vals_p, denom_t, bias.reshape(1, D))


# speedup vs baseline: 1.3086x; 1.0706x over previous
import functools

import jax
import jax.numpy as jnp
from jax import lax
from jax.experimental import pallas as pl
from jax.experimental.pallas import tpu as pltpu
from jax.experimental.pallas import tpu_sc as plsc

N = 10000
E = 320000
D = 128

NC = 2
NS = 16
NW = NC * NS
EP = E // NW
EC = 1000
CH = 40
NCHB = EC // CH
NQUAD = NCHB // 4
NGRP = EC // 16
REM = EC - NGRP * 16

ROWS_PER_TILE = 640


def _proj_body(x_ref, w_ref, a_ref, b_ref, sf_ref, f_ref):
    sf = jnp.dot(x_ref[...], w_ref[...], preferred_element_type=jnp.float32)
    sf_ref[...] = sf
    f_ref[...] = jnp.dot(sf, a_ref[...], preferred_element_type=jnp.float32) + b_ref[...]


def _tc_project(x, W, a_cat, b_cat):
    BR = 2000
    grid = (N // BR,)
    return pl.pallas_call(
        _proj_body,
        grid=grid,
        in_specs=[
            pl.BlockSpec((BR, D), lambda i: (i, 0)),
            pl.BlockSpec((D, D), lambda i: (0, 0)),
            pl.BlockSpec((D, 2), lambda i: (0, 0)),
            pl.BlockSpec((1, 2), lambda i: (0, 0)),
        ],
        out_specs=[
            pl.BlockSpec((BR, D), lambda i: (i, 0)),
            pl.BlockSpec((BR, 2), lambda i: (i, 0)),
        ],
        out_shape=[
            jax.ShapeDtypeStruct((N, D), jnp.float32),
            jax.ShapeDtypeStruct((N, 2), jnp.float32),
        ],
    )(x, W, a_cat, b_cat)


def _sc_edge_body(f12_hbm, src_hbm, dst_hbm, adj_hbm, sf_hbm,
                  denom_out, vals_out,
                  f12_v, src_v, dst_v, adj_v, src_w, dst_w, adj_w,
                  e_v, denom_v, rows_v,
                  vals_sh, gs0, gs1, gs2, gs3, ss0, ss1, ss2, ss3, stg):
    gss = [gs0, gs1, gs2, gs3]
    sss = [ss0, ss1, ss2, ss3]
    cid = lax.axis_index("c")
    sid = lax.axis_index("s")
    wid = sid * NC + cid
    base = wid * EP

    pltpu.sync_copy(f12_hbm, f12_v)

    zeros16 = jnp.zeros((16,), jnp.float32)

    def _zd(i, _):
        denom_v[pl.ds(i * 16, 16)] = zeros16
        return 0
    lax.fori_loop(0, N // 16, _zd, 0)

    def _zr(i, _):
        for b in range(4):
            for k in range(D // 16):
                rows_v[b, i, pl.ds(k * 16, 16)] = zeros16
        return 0
    lax.fori_loop(0, CH, _zr, 0)
    stripe = jnp.minimum(sid * ROWS_PER_TILE, N - ROWS_PER_TILE)
    for z in range(ROWS_PER_TILE // CH):
        pltpu.async_copy(rows_v.at[0],
                         vals_sh.at[pl.ds(stripe + z * CH, CH)], gs0)
    for z in range(ROWS_PER_TILE // CH):
        pltpu.make_async_copy(rows_v.at[0],
                              vals_sh.at[pl.ds(stripe + z * CH, CH)],
                              gs0).wait()
    plsc.subcore_barrier()

    def _stage(blk, sv, dv, av, sem):
        eb = base + blk * EC
        pltpu.async_copy(src_hbm.at[pl.ds(eb, EC)], sv, sem)
        pltpu.async_copy(dst_hbm.at[pl.ds(eb, EC)], dv, sem)
        pltpu.async_copy(adj_hbm.at[pl.ds(eb, EC)], av, sem)

    def _stage_wait(blk, sv, dv, av, sem):
        eb = base + blk * EC
        pltpu.make_async_copy(src_hbm.at[pl.ds(eb, EC)], sv, sem).wait()
        pltpu.make_async_copy(dst_hbm.at[pl.ds(eb, EC)], dv, sem).wait()
        pltpu.make_async_copy(adj_hbm.at[pl.ds(eb, EC)], av, sem).wait()

    def _process(blk, src_v, dst_v, adj_v):
        def _gidx(c):
            return dst_v.at[pl.ds(c * CH, CH)]

        def _sidx(c):
            return src_v.at[pl.ds(c * CH, CH)]

        def _gissue(c, b):
            pltpu.async_copy(sf_hbm.at[_gidx(c)], rows_v.at[b], gss[b])

        def _gwait(c, b):
            pltpu.make_async_copy(sf_hbm.at[_gidx(c)], rows_v.at[b],
                                  gss[b]).wait()

        def _sissue(c, b):
            pltpu.async_copy(rows_v.at[b], vals_sh.at[_sidx(c)], sss[b],
                             add=True)

        def _swait(c, b):
            pltpu.make_async_copy(rows_v.at[b], vals_sh.at[_sidx(c)],
                                  sss[b]).wait()

        _gissue(0, 0)
        _gissue(1, 1)

        def _e16(off, mask):
            s16 = src_v[pl.ds(off, 16)]
            d16 = dst_v[pl.ds(off, 16)]
            a16 = adj_v[pl.ds(off, 16)]
            ws = plsc.load_gather(f12_v, [s16])
            wd = plsc.load_gather(f12_v, [d16])
            f1s, _ = plsc.unpack(plsc.bitcast(ws, jnp.bfloat16),
                                 format=plsc.PackFormat.INTERLEAVED)
            _, f2d = plsc.unpack(plsc.bitcast(wd, jnp.bfloat16),
                                 format=plsc.PackFormat.INTERLEAVED)
            t = a16 * (f1s + f2d)
            lr = jnp.maximum(t, 0.2 * t)
            e16 = jnp.exp(lr)
            e_v[pl.ds(off, 16)] = e16
            plsc.addupdate_scatter(denom_v, [s16], e16, mask=mask)

        def _egrp(g, _):
            _e16(g * 16, None)
            return 0
        lax.fori_loop(0, NGRP, _egrp, 0)
        if REM:
            _e16(EC - 16, lax.iota(jnp.int32, 16) >= 16 - REM)

        def _proc(c, b):
            _gwait(c, b)

            def _grp(g, _):
                e16 = e_v[pl.ds(c * CH + g * 16, 16)]
                for j in range(16):
                    r = g * 16 + j
                    ej = e16[j]
                    for k in range(8):
                        rows_v[b, r, pl.ds(k * 16, 16)] = (
                            rows_v[b, r, pl.ds(k * 16, 16)] * ej)
                return 0
            lax.fori_loop(0, CH // 16, _grp, 0)
            e16 = e_v[pl.ds(c * CH + CH - 16, 16)]
            for j in range(CH - (CH // 16) * 16):
                r = (CH // 16) * 16 + j
                ej = e16[16 - (CH - (CH // 16) * 16) + j]
                for k in range(8):
                    rows_v[b, r, pl.ds(k * 16, 16)] = (
                        rows_v[b, r, pl.ds(k * 16, 16)] * ej)

            _sissue(c, b)

        def _quad(t, _):
            for u in range(4):
                c = 4 * t + u
                pb = (u + 2) % 4
                if u in (0, 1):
                    @pl.when(t > 0)
                    def _():
                        _swait(c - 2, pb)
                    _gissue(c + 2, pb)
                elif u == 2:
                    _swait(c - 2, pb)
                    _gissue(c + 2, pb)
                else:
                    @pl.when(t < NQUAD - 1)
                    def _():
                        _swait(c - 2, pb)
                        _gissue(c + 2, pb)
                _proc(c, u)
            return 0
        lax.fori_loop(0, NQUAD, _quad, 0)
        _proc(NCHB - 1, 0)
        _swait(NCHB - 4, 1)
        _swait(NCHB - 3, 2)
        _swait(NCHB - 2, 3)
        _swait(NCHB - 1, 0)

    NSUP = EP // EC // 2
    _stage(0, src_v, dst_v, adj_v, stg)
    _stage_wait(0, src_v, dst_v, adj_v, stg)

    def _super(s, _):
        a = 2 * s
        _stage(a + 1, src_w, dst_w, adj_w, stg)
        _process(a, src_v, dst_v, adj_v)
        _stage_wait(a + 1, src_w, dst_w, adj_w, stg)

        @pl.when(s < NSUP - 1)
        def _():
            _stage(a + 2, src_v, dst_v, adj_v, stg)
        _process(a + 1, src_w, dst_w, adj_w)

        @pl.when(s < NSUP - 1)
        def _():
            _stage_wait(a + 2, src_v, dst_v, adj_v, stg)
        return 0
    lax.fori_loop(0, NSUP, _super, 0)

    plsc.subcore_barrier()
    pltpu.sync_copy(denom_v, denom_out.at[pl.ds(wid * N, N)])
    pltpu.sync_copy(vals_sh.at[pl.ds(stripe, ROWS_PER_TILE)],
                    vals_out.at[cid, pl.ds(stripe, ROWS_PER_TILE)])


def _sc_edge(f12, src, dst, adj, sf):
    mesh = plsc.VectorSubcoreMesh(core_axis_name="c", subcore_axis_name="s",
                                  num_cores=NC)
    fn = functools.partial(
        pl.kernel,
        mesh=mesh,
        compiler_params=pltpu.CompilerParams(needs_layout_passes=False),
        out_type=[
            jax.ShapeDtypeStruct((NW * N,), jnp.float32),
            jax.ShapeDtypeStruct((NC, N, D), jnp.float32),
        ],
        scratch_types=[
            pltpu.VMEM((N,), jnp.int32),
            pltpu.VMEM((EC,), jnp.int32),
            pltpu.VMEM((EC,), jnp.int32),
            pltpu.VMEM((EC,), jnp.float32),
            pltpu.VMEM((EC,), jnp.int32),
            pltpu.VMEM((EC,), jnp.int32),
            pltpu.VMEM((EC,), jnp.float32),
            pltpu.VMEM((EC,), jnp.float32),
            pltpu.VMEM((N,), jnp.float32),
            pltpu.VMEM((4, CH, D), jnp.float32),
            pltpu.VMEM_SHARED((N, D), jnp.float32),
            pltpu.SemaphoreType.DMA,
            pltpu.SemaphoreType.DMA,
            pltpu.SemaphoreType.DMA,
            pltpu.SemaphoreType.DMA,
            pltpu.SemaphoreType.DMA,
            pltpu.SemaphoreType.DMA,
            pltpu.SemaphoreType.DMA,
            pltpu.SemaphoreType.DMA,
            pltpu.SemaphoreType.DMA,
        ],
    )(_sc_edge_body)
    return fn(f12, src, dst, adj, sf)


def _final_body(vals_ref, denom_ref, bias_ref, o_ref):
    v = vals_ref[0] + vals_ref[1]
    d = jnp.sum(denom_ref[...], axis=1)
    d = jnp.where(d > 0.0, d, 1.0)
    r = v / d[:, None] + bias_ref[...]
    o_ref[...] = jnp.where(r > 0.0, r, jnp.exp(jnp.minimum(r, 0.0)) - 1.0)


def _tc_finalize(vals_p, denom_t, bias):
    BR = 2000
    return pl.pallas_call(
        _final_body,
        grid=(N // BR,),
        in_specs=[
            pl.BlockSpec((NC, BR, D), lambda i: (0, i, 0)),
            pl.BlockSpec((BR, NW), lambda i: (i, 0)),
            pl.BlockSpec((1, D), lambda i: (0, 0)),
        ],
        out_specs=pl.BlockSpec((BR, D), lambda i: (i, 0)),
        out_shape=jax.ShapeDtypeStruct((N, D), jnp.float32),
    )(vals_p, denom_t, bias)


@jax.jit
def kernel(x, edge_index, adj_vals, W, a1, b1, a2, b2, bias):
    src = edge_index[0]
    dst = edge_index[1]
    a_cat = jnp.concatenate([a1, a2], axis=1)
    b_cat = jnp.stack([b1[0], b2[0]])[None, :]
    sf, f = _tc_project(x, W, a_cat, b_cat)
    f12 = jax.lax.bitcast_convert_type(f.astype(jnp.bfloat16), jnp.int32)
    denom_p, vals_p = _sc_edge(f12, src, dst, adj_vals, sf)
    denom_t = denom_p.reshape(NW, N).T
    return _tc_finalize(vals_p, denom_t, bias.reshape(1, D))
